# baseline (speedup 1.0000x reference)
import jax
import jax.numpy as jnp
from jax.experimental import pallas as pl

_MAX_FREQUENCY = 128.0
_NUM_TOTAL_FREQUENCIES = 128
_BN = 2048

_COS_C = (0.9989871016246259, -19.591096382371575, 61.5970720980049,
          -61.08884330070406)


def _pp2d_block(x_ref, pos_ref, scl_ref, rot_ref, coef_ref, idx_ref,
                col_ref, out_ref):
    xb = x_ref[...]
    x0 = xb[:, 0:1]
    x1 = xb[:, 1:2]
    pos = pos_ref[...].T
    scl = scl_ref[...].T
    rot = rot_ref[...].T
    coef = coef_ref[...].T
    freq = (idx_ref[...].astype(jnp.float32)
            * (_MAX_FREQUENCY / _NUM_TOTAL_FREQUENCIES)).T
    c = jnp.cos(rot)
    s = jnp.sin(rot)
    sx = scl[0:1, :]
    sy = scl[1:2, :]
    v1 = -s * sy
    v2 = c * sy
    dx = x0 - pos[0:1, :]
    dy = x1 - pos[1:2, :]
    tx = c * dx + s * dy
    gx = tx * sx
    gy = v1 * dx + v2 * dy
    env = jnp.exp(-0.5 * (gx * gx + gy * gy))
    wave = None
    for k in range(coef.shape[0]):
        fk = freq[k:k + 1, :]
        ck = coef[k:k + 1, :]
        p = fk * tx
        u = p - jnp.round(p)
        t = u * u
        acc = ck * jnp.float32(_COS_C[-1])
        for a in _COS_C[-2::-1]:
            acc = acc * t + ck * jnp.float32(a)
        wave = acc if wave is None else wave + acc
    out_ref[...] = jnp.dot(env * wave, col_ref[...],
                           preferred_element_type=jnp.float32)


def kernel(x, gaussian_colors, gaussian_positions, gaussian_scales,
           gaussian_rotations, topk_wave_coefficients, topk_wave_indices):
    n, _ = x.shape
    g, num_out = gaussian_colors.shape
    k = topk_wave_coefficients.shape[1]
    idx = topk_wave_indices.astype(jnp.int32)

    grid = (n // _BN,)
    out = pl.pallas_call(
        _pp2d_block,
        grid=grid,
        in_specs=[
            pl.BlockSpec((_BN, 2), lambda i: (i, 0)),
            pl.BlockSpec((g, 2), lambda i: (0, 0)),
            pl.BlockSpec((g, 2), lambda i: (0, 0)),
            pl.BlockSpec((g, 1), lambda i: (0, 0)),
            pl.BlockSpec((g, k), lambda i: (0, 0)),
            pl.BlockSpec((g, k), lambda i: (0, 0)),
            pl.BlockSpec((g, num_out), lambda i: (0, 0)),
        ],
        out_specs=pl.BlockSpec((_BN, num_out), lambda i: (i, 0)),
        out_shape=jax.ShapeDtypeStruct((n, num_out), jnp.float32),
    )(x, gaussian_positions, gaussian_scales, gaussian_rotations,
      topk_wave_coefficients, idx, gaussian_colors)
    return out


# speedup vs baseline: 1.1293x; 1.1293x over previous
import jax
import jax.numpy as jnp
from jax.experimental import pallas as pl

_MAX_FREQUENCY = 128.0
_NUM_TOTAL_FREQUENCIES = 128
_BN = 2048

_NEG_HALF_LOG2E = -0.7213475204444817

_COS_C = (0.9989871016246259, -19.591096382371575, 61.5970720980049,
          -61.08884330070406)


def _pp2d_block(x_ref, pos_ref, scl_ref, rot_ref, coef_ref, freq_ref,
                col_ref, out_ref):
    xb = x_ref[...]
    x0 = xb[:, 0:1]
    x1 = xb[:, 1:2]
    pos = pos_ref[...]
    scl = scl_ref[...]
    rot = rot_ref[...]
    c = jnp.cos(rot)
    s = jnp.sin(rot)
    sx = scl[0:1, :]
    sy = scl[1:2, :]
    v1 = -s * sy
    v2 = c * sy
    dx = x0 - pos[0:1, :]
    dy = x1 - pos[1:2, :]
    tx = c * dx + s * dy
    gx = tx * sx
    gy = v1 * dx + v2 * dy
    env = jnp.exp2((gx * gx + gy * gy) * jnp.float32(_NEG_HALF_LOG2E))
    wave = None
    for k in range(freq_ref.shape[0]):
        fk = freq_ref[k:k + 1, :]
        ck = coef_ref[k:k + 1, :]
        p = fk * tx
        u = p - jnp.round(p)
        t = u * u
        acc = ck * jnp.float32(_COS_C[-1])
        for a in _COS_C[-2::-1]:
            acc = acc * t + ck * jnp.float32(a)
        wave = acc if wave is None else wave + acc
    out_ref[...] = jnp.dot(env * wave, col_ref[...],
                           preferred_element_type=jnp.float32)


def kernel(x, gaussian_colors, gaussian_positions, gaussian_scales,
           gaussian_rotations, topk_wave_coefficients, topk_wave_indices):
    n, _ = x.shape
    g, num_out = gaussian_colors.shape
    k = topk_wave_coefficients.shape[1]
    freqs = (topk_wave_indices.astype(jnp.float32)
             * (_MAX_FREQUENCY / _NUM_TOTAL_FREQUENCIES)).T
    coefs = topk_wave_coefficients.T
    pos_t = gaussian_positions.T
    scl_t = gaussian_scales.T
    rot_t = gaussian_rotations.T

    grid = (n // _BN,)
    out = pl.pallas_call(
        _pp2d_block,
        grid=grid,
        in_specs=[
            pl.BlockSpec((_BN, 2), lambda i: (i, 0)),
            pl.BlockSpec((2, g), lambda i: (0, 0)),
            pl.BlockSpec((2, g), lambda i: (0, 0)),
            pl.BlockSpec((1, g), lambda i: (0, 0)),
            pl.BlockSpec((k, g), lambda i: (0, 0)),
            pl.BlockSpec((k, g), lambda i: (0, 0)),
            pl.BlockSpec((g, num_out), lambda i: (0, 0)),
        ],
        out_specs=pl.BlockSpec((_BN, num_out), lambda i: (i, 0)),
        out_shape=jax.ShapeDtypeStruct((n, num_out), jnp.float32),
    )(x, pos_t, scl_t, rot_t, coefs, freqs, gaussian_colors)
    return out
